# baseline (speedup 1.0000x reference)
import functools

import jax
import jax.numpy as jnp
from jax import lax
from jax.experimental import pallas as pl
from jax.experimental.pallas import tpu as pltpu
from jax.experimental.pallas import tpu_sc as plsc

LANES = 16
NPB = 8


def _qp_tc(emb, aw2, bhalf):
    rows, h = emb.shape
    blk = 1024
    grid = (rows + blk - 1) // blk

    def body(emb_ref, aw_ref, b_ref, out_ref):
        out_ref[...] = lax.dot_general(
            aw_ref[...], emb_ref[...],
            dimension_numbers=(((1,), (1,)), ((), ())),
            preferred_element_type=jnp.float32,
        ) + b_ref[0]

    return pl.pallas_call(
        body,
        grid=(grid,),
        in_specs=[
            pl.BlockSpec((blk, h), lambda i: (i, 0)),
            pl.BlockSpec((2, h), lambda i: (0, 0)),
            pl.BlockSpec(memory_space=pltpu.SMEM),
        ],
        out_specs=pl.BlockSpec((2, blk), lambda i: (0, i)),
        out_shape=jax.ShapeDtypeStruct((2, rows), jnp.float32),
    )(emb, aw2, bhalf)


def _gat_sc(nid_flat, neigh_b, mask_b, emb, q1, p1, m):
    nt = nid_flat.shape[0]
    h = emb.shape[1]
    hc = h // LANES
    nb_rows_per_block = NPB * m
    g_rows = nb_rows_per_block // 128

    nc, ns = 2, 16
    nw = nc * ns
    nblocks = nt // NPB
    blocks_per_w = nblocks // nw
    mesh = plsc.VectorSubcoreMesh(core_axis_name="c", subcore_axis_name="s",
                                  num_cores=nc, num_subcores=ns)

    buf_types = [
        pltpu.VMEM((NPB,), jnp.int32),
        pltpu.VMEM((g_rows, 128), jnp.int32),
        pltpu.VMEM((g_rows, 128), jnp.float32),
        pltpu.VMEM((NPB, h), jnp.float32),
        pltpu.VMEM((g_rows, 128, h), jnp.float32),
        pltpu.VMEM((LANES,), jnp.float32),
        pltpu.VMEM((LANES,), jnp.float32),
        pltpu.VMEM((g_rows, 128), jnp.float32),
        pltpu.VMEM((NPB, h), jnp.float32),
        pltpu.SemaphoreType.DMA,
        pltpu.SemaphoreType.DMA,
    ]

    @functools.partial(
        pl.kernel,
        out_type=jax.ShapeDtypeStruct((nt, h), jnp.float32),
        mesh=mesh,
        scratch_types=[buf_types, buf_types],
    )
    def k(nid_hbm, neigh_hbm, mask_hbm, emb_hbm, q_hbm, p_hbm, out_hbm,
          buf_a, buf_b):
        wid = lax.axis_index("s") * nc + lax.axis_index("c")
        iota = lax.iota(jnp.int32, LANES)
        wbase = wid * blocks_per_w

        def _shuf(x, sh):
            return x.at[iota ^ sh].get(mode="promise_in_bounds")

        def allmax(x):
            for sh in (8, 4, 2, 1):
                x = jnp.maximum(x, _shuf(x, sh))
            return x

        def allsum(x):
            for sh in (8, 4, 2, 1):
                x = x + _shuf(x, sh)
            return x

        def issue(blk, buf):
            (nid_v, nbr_v, mask_v, src_rows, nb_rows, qsrc_v, psrc_v,
             pnb_v, out_v, sem, out_sem) = buf
            blk = jnp.minimum(blk, nblocks - 1)
            nbase = blk * NPB
            pltpu.sync_copy(nid_hbm.at[pl.ds(nbase, NPB)], nid_v)
            pltpu.sync_copy(neigh_hbm.at[blk], nbr_v)
            pltpu.sync_copy(mask_hbm.at[blk], mask_v)
            cps = [
                pltpu.async_copy(emb_hbm.at[nid_v], src_rows, sem),
                pltpu.async_copy(q_hbm.at[nid_v], qsrc_v.at[pl.ds(0, NPB)], sem),
                pltpu.async_copy(p_hbm.at[nid_v], psrc_v.at[pl.ds(0, NPB)], sem),
            ]
            for g in range(g_rows):
                cps.append(pltpu.async_copy(emb_hbm.at[nbr_v.at[g]],
                                            nb_rows.at[g], sem))
                cps.append(pltpu.async_copy(p_hbm.at[nbr_v.at[g]],
                                            pnb_v.at[g], sem))
            return cps

        def wait(cps):
            for cp in cps:
                cp.wait()

        def compute(blk, buf, first):
            (nid_v, nbr_v, mask_v, src_rows, nb_rows, qsrc_v, psrc_v,
             pnb_v, out_v, sem, out_sem) = buf
            nbase = blk * NPB
            qv = qsrc_v[...]
            pv = psrc_v[...]

            @pl.when(jnp.logical_not(first))
            def _():
                pltpu.make_async_copy(
                    out_v, out_hbm.at[pl.ds(nbase, NPB)], out_sem).wait()

            def node_body(n, _):
                nfull = jnp.full((LANES,), n, jnp.int32)
                q_s = qv.at[nfull].get(mode="promise_in_bounds")
                p_s = pv.at[nfull].get(mode="promise_in_bounds")
                s_self = q_s + p_s
                s_self = jnp.where(s_self >= 0, s_self, 0.2 * s_self)

                svecs = []
                for gidx in range(m // LANES):
                    flat = n * m + gidx * LANES
                    grow = flat // 128
                    roff = pl.multiple_of(flat % 128, LANES)
                    p_nb = pnb_v[grow, pl.ds(roff, LANES)]
                    s = q_s + p_nb
                    s = jnp.where(s >= 0, s, 0.2 * s)
                    msk = mask_v[grow, pl.ds(roff, LANES)]
                    svecs.append(s + msk * (-1e9))

                smax = jnp.maximum(svecs[0], svecs[1])
                smax = jnp.maximum(smax, s_self)
                mval = allmax(smax)
                e0 = jnp.exp(svecs[0] - mval)
                e1 = jnp.exp(svecs[1] - mval)
                e_self = jnp.exp(s_self - mval)
                e_self_one = jnp.where(iota == 0, e_self, 0.0)
                denom = allsum(e0 + e1 + e_self_one)
                inv = 1.0 / denom
                ws = (e0 * inv, e1 * inv)
                w_self = e_self * inv

                acc = tuple(
                    w_self * src_rows[n, pl.ds(c * LANES, LANES)]
                    for c in range(hc))

                for gidx in range(m // LANES):
                    wg = ws[gidx]

                    def nb_body(j, acc, gidx=gidx, wg=wg):
                        w_j = wg.at[jnp.full((LANES,), j, jnp.int32)].get(
                            mode="promise_in_bounds")
                        flat = n * m + gidx * LANES + j
                        grow = flat // 128
                        roff = flat % 128
                        return tuple(
                            acc[c] + w_j * nb_rows[grow, roff,
                                                   pl.ds(c * LANES, LANES)]
                            for c in range(hc))

                    acc = lax.fori_loop(0, LANES, nb_body, acc, unroll=2)
                for c in range(hc):
                    out_v[n, pl.ds(c * LANES, LANES)] = acc[c]
                return 0

            lax.fori_loop(0, NPB, node_body, 0)
            pltpu.async_copy(out_v, out_hbm.at[pl.ds(nbase, NPB)], out_sem)

        cps_a = issue(wbase, buf_a)
        cps_b = issue(wbase + 1, buf_b)

        def loop_body(i, _):
            ba = wbase + 2 * i
            wait(cps_a)
            compute(ba, buf_a, i == 0)
            issue(ba + 2, buf_a)
            wait(cps_b)
            compute(ba + 1, buf_b, i == 0)
            issue(ba + 3, buf_b)
            return 0

        lax.fori_loop(0, blocks_per_w // 2, loop_body, 0)
        wait(cps_a)
        wait(cps_b)
        last_a = wbase + blocks_per_w - 2
        last_b = wbase + blocks_per_w - 1
        pltpu.make_async_copy(
            buf_a[8], out_hbm.at[pl.ds(last_a * NPB, NPB)], buf_a[10]).wait()
        pltpu.make_async_copy(
            buf_b[8], out_hbm.at[pl.ds(last_b * NPB, NPB)], buf_b[10]).wait()

    return k(nid_flat, neigh_b, mask_b, emb, q1, p1)


def kernel(node_ids, neighs, mask, emb_table, a_w, a_b):
    b, l = node_ids.shape
    m = neighs.shape[-1]
    h = emb_table.shape[1]
    nt = b * l

    aw2 = a_w.reshape(2, h)
    bhalf = (a_b * 0.5).astype(jnp.float32)
    qp2 = _qp_tc(emb_table.astype(jnp.float32), aw2.astype(jnp.float32),
                 bhalf)
    q1 = qp2[0]
    p1 = qp2[1]

    nid_flat = node_ids.reshape(nt).astype(jnp.int32)
    gb = (NPB * m) // 128
    neigh_b = neighs.reshape(nt // NPB, gb, 128).astype(jnp.int32)
    mask_b = mask.reshape(nt // NPB, gb, 128).astype(jnp.float32)

    out = _gat_sc(nid_flat, neigh_b, mask_b, emb_table.astype(jnp.float32),
                  q1, p1, m)
    return out.reshape(b, l, h)


# speedup vs baseline: 1.2812x; 1.2812x over previous
import functools

import jax
import jax.numpy as jnp
from jax import lax
from jax.experimental import pallas as pl
from jax.experimental.pallas import tpu as pltpu
from jax.experimental.pallas import tpu_sc as plsc

LANES = 16
NPB = 8


def _qp_tc(emb, aw2, bhalf):
    rows, h = emb.shape
    blk = 1024
    grid = (rows + blk - 1) // blk

    def body(emb_ref, aw_ref, b_ref, out_ref):
        out_ref[...] = lax.dot_general(
            aw_ref[...], emb_ref[...],
            dimension_numbers=(((1,), (1,)), ((), ())),
            preferred_element_type=jnp.float32,
        ) + b_ref[0]

    return pl.pallas_call(
        body,
        grid=(grid,),
        in_specs=[
            pl.BlockSpec((blk, h), lambda i: (i, 0)),
            pl.BlockSpec((2, h), lambda i: (0, 0)),
            pl.BlockSpec(memory_space=pltpu.SMEM),
        ],
        out_specs=pl.BlockSpec((2, blk), lambda i: (0, i)),
        out_shape=jax.ShapeDtypeStruct((2, rows), jnp.float32),
    )(emb, aw2, bhalf)


def _gat_sc(nid_flat, neigh_b, mask_b, emb, q1, p1, m):
    nt = nid_flat.shape[0]
    h = emb.shape[1]
    hc = h // LANES
    nb_rows_per_block = NPB * m
    g_rows = nb_rows_per_block // 128

    nc, ns = 2, 16
    nw = nc * ns
    nblocks = nt // NPB
    blocks_per_w = nblocks // nw
    mesh = plsc.VectorSubcoreMesh(core_axis_name="c", subcore_axis_name="s",
                                  num_cores=nc, num_subcores=ns)

    idx_types = [
        pltpu.VMEM((NPB,), jnp.int32),
        pltpu.VMEM((g_rows, 128), jnp.int32),
        pltpu.VMEM((g_rows, 128), jnp.float32),
        pltpu.SemaphoreType.DMA,
    ]
    data_types = [
        pltpu.VMEM((NPB, h), jnp.float32),
        pltpu.VMEM((g_rows, 128, h), jnp.float32),
        pltpu.VMEM((LANES,), jnp.float32),
        pltpu.VMEM((LANES,), jnp.float32),
        pltpu.VMEM((g_rows, 128), jnp.float32),
        pltpu.VMEM((NPB, h), jnp.float32),
        pltpu.SemaphoreType.DMA,
        pltpu.SemaphoreType.DMA,
    ]

    @functools.partial(
        pl.kernel,
        out_type=jax.ShapeDtypeStruct((nt, h), jnp.float32),
        mesh=mesh,
        scratch_types=[[idx_types] * 4, [data_types] * 2],
    )
    def k(nid_hbm, neigh_hbm, mask_hbm, emb_hbm, q_hbm, p_hbm, out_hbm,
          xsets, dsets):
        wid = lax.axis_index("s") * nc + lax.axis_index("c")
        iota = lax.iota(jnp.int32, LANES)
        wbase = wid * blocks_per_w

        def _shuf(x, sh):
            return x.at[iota ^ sh].get(mode="promise_in_bounds")

        def allmax(x):
            for sh in (8, 4, 2, 1):
                x = jnp.maximum(x, _shuf(x, sh))
            return x

        def allsum(x):
            for sh in (8, 4, 2, 1):
                x = x + _shuf(x, sh)
            return x

        def idx_fire(blk, xs):
            nid_v, nbr_v, mask_v, isem = xs
            blk = jnp.minimum(blk, nblocks - 1)
            nbase = blk * NPB
            pltpu.async_copy(nid_hbm.at[pl.ds(nbase, NPB)], nid_v, isem)
            pltpu.async_copy(neigh_hbm.at[blk], nbr_v, isem)
            pltpu.async_copy(mask_hbm.at[blk], mask_v, isem)

        def idx_wait(xs):
            nid_v, nbr_v, mask_v, isem = xs
            pltpu.make_async_copy(nid_hbm.at[pl.ds(0, NPB)], nid_v, isem).wait()
            pltpu.make_async_copy(neigh_hbm.at[0], nbr_v, isem).wait()
            pltpu.make_async_copy(mask_hbm.at[0], mask_v, isem).wait()

        def g_fire(xs, ds):
            nid_v, nbr_v, mask_v, isem = xs
            (src_rows, nb_rows, qsrc_v, psrc_v, pnb_v, out_v, gsem,
             osem) = ds
            pltpu.async_copy(emb_hbm.at[nid_v], src_rows, gsem)
            pltpu.async_copy(q_hbm.at[nid_v], qsrc_v.at[pl.ds(0, NPB)], gsem)
            pltpu.async_copy(p_hbm.at[nid_v], psrc_v.at[pl.ds(0, NPB)], gsem)
            for g in range(g_rows):
                pltpu.async_copy(emb_hbm.at[nbr_v.at[g]], nb_rows.at[g], gsem)
                pltpu.async_copy(p_hbm.at[nbr_v.at[g]], pnb_v.at[g], gsem)

        def g_wait(xs, ds):
            nid_v, nbr_v, mask_v, isem = xs
            (src_rows, nb_rows, qsrc_v, psrc_v, pnb_v, out_v, gsem,
             osem) = ds
            pltpu.make_async_copy(emb_hbm.at[nid_v], src_rows, gsem).wait()
            pltpu.make_async_copy(
                q_hbm.at[nid_v], qsrc_v.at[pl.ds(0, NPB)], gsem).wait()
            pltpu.make_async_copy(
                p_hbm.at[nid_v], psrc_v.at[pl.ds(0, NPB)], gsem).wait()
            for g in range(g_rows):
                pltpu.make_async_copy(
                    emb_hbm.at[nbr_v.at[g]], nb_rows.at[g], gsem).wait()
                pltpu.make_async_copy(
                    p_hbm.at[nbr_v.at[g]], pnb_v.at[g], gsem).wait()

        def out_drain(blk, ds):
            (src_rows, nb_rows, qsrc_v, psrc_v, pnb_v, out_v, gsem,
             osem) = ds
            pltpu.make_async_copy(
                out_v, out_hbm.at[pl.ds(blk * NPB, NPB)], osem).wait()

        def compute(blk, ds, xs, first):
            nid_v, nbr_v, mask_v, isem = xs
            (src_rows, nb_rows, qsrc_v, psrc_v, pnb_v, out_v, gsem,
             osem) = ds
            nbase = blk * NPB
            qv = qsrc_v[...]
            pv = psrc_v[...]

            @pl.when(jnp.logical_not(first))
            def _():
                out_drain(blk, ds)

            def node_body(n, _):
                nfull = jnp.full((LANES,), n, jnp.int32)
                q_s = qv.at[nfull].get(mode="promise_in_bounds")
                p_s = pv.at[nfull].get(mode="promise_in_bounds")
                s_self = q_s + p_s
                s_self = jnp.where(s_self >= 0, s_self, 0.2 * s_self)

                svecs = []
                for gidx in range(m // LANES):
                    flat = n * m + gidx * LANES
                    grow = flat // 128
                    roff = pl.multiple_of(flat % 128, LANES)
                    p_nb = pnb_v[grow, pl.ds(roff, LANES)]
                    s = q_s + p_nb
                    s = jnp.where(s >= 0, s, 0.2 * s)
                    msk = mask_v[grow, pl.ds(roff, LANES)]
                    svecs.append(s + msk * (-1e9))

                smax = jnp.maximum(svecs[0], svecs[1])
                smax = jnp.maximum(smax, s_self)
                mval = allmax(smax)
                e0 = jnp.exp(svecs[0] - mval)
                e1 = jnp.exp(svecs[1] - mval)
                e_self = jnp.exp(s_self - mval)
                e_self_one = jnp.where(iota == 0, e_self, 0.0)
                denom = allsum(e0 + e1 + e_self_one)
                inv = 1.0 / denom
                ws = (e0 * inv, e1 * inv)
                w_self = e_self * inv

                acc = tuple(
                    w_self * src_rows[n, pl.ds(c * LANES, LANES)]
                    for c in range(hc))

                for gidx in range(m // LANES):
                    wg = ws[gidx]

                    def nb_body(j, acc, gidx=gidx, wg=wg):
                        w_j = wg.at[jnp.full((LANES,), j, jnp.int32)].get(
                            mode="promise_in_bounds")
                        flat = n * m + gidx * LANES + j
                        grow = flat // 128
                        roff = flat % 128
                        return tuple(
                            acc[c] + w_j * nb_rows[grow, roff,
                                                   pl.ds(c * LANES, LANES)]
                            for c in range(hc))

                    acc = lax.fori_loop(0, LANES, nb_body, acc, unroll=2)
                for c in range(hc):
                    out_v[n, pl.ds(c * LANES, LANES)] = acc[c]
                return 0

            lax.fori_loop(0, NPB, node_body, 0)
            pltpu.async_copy(out_v, out_hbm.at[pl.ds(nbase, NPB)], osem)

        for kk in range(4):
            idx_fire(wbase + kk, xsets[kk])
        idx_wait(xsets[0])
        g_fire(xsets[0], dsets[0])
        idx_wait(xsets[1])
        g_fire(xsets[1], dsets[1])

        def loop_body(i, _):
            b0 = wbase + 4 * i
            for k2 in range(4):
                ds = dsets[k2 % 2]
                xc = xsets[k2]
                xn = xsets[(k2 + 2) % 4]
                blk = b0 + k2
                g_wait(xc, ds)
                first = (i == 0) if k2 < 2 else (i < 0)
                compute(blk, ds, xc, first)
                idx_wait(xn)
                g_fire(xn, ds)
                idx_fire(blk + 4, xc)
            return 0

        lax.fori_loop(0, blocks_per_w // 4, loop_body, 0)

        g_wait(xsets[2], dsets[0])
        g_wait(xsets[3], dsets[1])
        idx_wait(xsets[2])
        idx_wait(xsets[3])
        out_drain(wbase + blocks_per_w - 2, dsets[0])
        out_drain(wbase + blocks_per_w - 1, dsets[1])

    return k(nid_flat, neigh_b, mask_b, emb, q1, p1)


def kernel(node_ids, neighs, mask, emb_table, a_w, a_b):
    b, l = node_ids.shape
    m = neighs.shape[-1]
    h = emb_table.shape[1]
    nt = b * l

    aw2 = a_w.reshape(2, h)
    bhalf = (a_b * 0.5).astype(jnp.float32)
    qp2 = _qp_tc(emb_table.astype(jnp.float32), aw2.astype(jnp.float32),
                 bhalf)
    q1 = qp2[0]
    p1 = qp2[1]

    nid_flat = node_ids.reshape(nt).astype(jnp.int32)
    gb = (NPB * m) // 128
    neigh_b = neighs.reshape(nt // NPB, gb, 128).astype(jnp.int32)
    mask_b = mask.reshape(nt // NPB, gb, 128).astype(jnp.float32)

    out = _gat_sc(nid_flat, neigh_b, mask_b, emb_table.astype(jnp.float32),
                  q1, p1, m)
    return out.reshape(b, l, h)


# speedup vs baseline: 1.5013x; 1.1718x over previous
import functools

import jax
import jax.numpy as jnp
from jax import lax
from jax.experimental import pallas as pl
from jax.experimental.pallas import tpu as pltpu
from jax.experimental.pallas import tpu_sc as plsc

LANES = 16
NPB = 8


def _qp_tc(emb, aw2, bhalf):
    rows, h = emb.shape
    blk = 4096
    grid = (rows + blk - 1) // blk

    def body(emb_ref, aw_ref, b_ref, out_ref):
        out_ref[...] = lax.dot_general(
            aw_ref[...], emb_ref[...],
            dimension_numbers=(((1,), (1,)), ((), ())),
            preferred_element_type=jnp.float32,
        ) + b_ref[0]

    return pl.pallas_call(
        body,
        grid=(grid,),
        in_specs=[
            pl.BlockSpec((blk, h), lambda i: (i, 0)),
            pl.BlockSpec((2, h), lambda i: (0, 0)),
            pl.BlockSpec(memory_space=pltpu.SMEM),
        ],
        out_specs=pl.BlockSpec((2, blk), lambda i: (0, i)),
        out_shape=jax.ShapeDtypeStruct((2, rows), jnp.float32),
    )(emb, aw2, bhalf)


def _gat_sc(nid_flat, neigh_b, mask_b, emb, q1, p1, m):
    nt = nid_flat.shape[0]
    h = emb.shape[1]
    hc = h // LANES
    nb_rows_per_block = NPB * m
    g_rows = nb_rows_per_block // 128

    nc, ns = 2, 16
    nw = nc * ns
    nblocks = nt // NPB
    blocks_per_w = nblocks // nw
    mesh = plsc.VectorSubcoreMesh(core_axis_name="c", subcore_axis_name="s",
                                  num_cores=nc, num_subcores=ns)

    idx_types = [
        pltpu.VMEM((NPB,), jnp.int32),
        pltpu.VMEM((g_rows, 128), jnp.int32),
        pltpu.VMEM((g_rows, 128), jnp.float32),
        pltpu.SemaphoreType.DMA,
    ]
    data_types = [
        pltpu.VMEM((NPB, h), jnp.float32),
        pltpu.VMEM((g_rows, 128, h), jnp.float32),
        pltpu.VMEM((LANES,), jnp.float32),
        pltpu.VMEM((LANES,), jnp.float32),
        pltpu.VMEM((g_rows, 128), jnp.float32),
        pltpu.VMEM((NPB, h), jnp.float32),
        pltpu.SemaphoreType.DMA,
        pltpu.SemaphoreType.DMA,
    ]

    @functools.partial(
        pl.kernel,
        out_type=jax.ShapeDtypeStruct((nt, h), jnp.float32),
        mesh=mesh,
        scratch_types=[[idx_types] * 4, [data_types] * 2],
    )
    def k(nid_hbm, neigh_hbm, mask_hbm, emb_hbm, q_hbm, p_hbm, out_hbm,
          xsets, dsets):
        wid = lax.axis_index("s") * nc + lax.axis_index("c")
        iota = lax.iota(jnp.int32, LANES)
        wbase = wid * blocks_per_w

        def _shuf(x, sh):
            return x.at[iota ^ sh].get(mode="promise_in_bounds")

        def allmax(x):
            for sh in (8, 4, 2, 1):
                x = jnp.maximum(x, _shuf(x, sh))
            return x

        def allsum(x):
            for sh in (8, 4, 2, 1):
                x = x + _shuf(x, sh)
            return x

        def idx_fire(blk, xs):
            nid_v, nbr_v, mask_v, isem = xs
            blk = jnp.minimum(blk, nblocks - 1)
            nbase = blk * NPB
            pltpu.async_copy(nid_hbm.at[pl.ds(nbase, NPB)], nid_v, isem)
            pltpu.async_copy(neigh_hbm.at[blk], nbr_v, isem)
            pltpu.async_copy(mask_hbm.at[blk], mask_v, isem)

        def idx_wait(xs):
            nid_v, nbr_v, mask_v, isem = xs
            pltpu.make_async_copy(nid_hbm.at[pl.ds(0, NPB)], nid_v, isem).wait()
            pltpu.make_async_copy(neigh_hbm.at[0], nbr_v, isem).wait()
            pltpu.make_async_copy(mask_hbm.at[0], mask_v, isem).wait()

        def g_fire(xs, ds):
            nid_v, nbr_v, mask_v, isem = xs
            (src_rows, nb_rows, qsrc_v, psrc_v, pnb_v, out_v, gsem,
             osem) = ds
            pltpu.async_copy(emb_hbm.at[nid_v], src_rows, gsem)
            pltpu.async_copy(q_hbm.at[nid_v], qsrc_v.at[pl.ds(0, NPB)], gsem)
            pltpu.async_copy(p_hbm.at[nid_v], psrc_v.at[pl.ds(0, NPB)], gsem)
            for g in range(g_rows):
                pltpu.async_copy(emb_hbm.at[nbr_v.at[g]], nb_rows.at[g], gsem)
                pltpu.async_copy(p_hbm.at[nbr_v.at[g]], pnb_v.at[g], gsem)

        def g_wait(xs, ds):
            nid_v, nbr_v, mask_v, isem = xs
            (src_rows, nb_rows, qsrc_v, psrc_v, pnb_v, out_v, gsem,
             osem) = ds
            pltpu.make_async_copy(emb_hbm.at[nid_v], src_rows, gsem).wait()
            pltpu.make_async_copy(
                q_hbm.at[nid_v], qsrc_v.at[pl.ds(0, NPB)], gsem).wait()
            pltpu.make_async_copy(
                p_hbm.at[nid_v], psrc_v.at[pl.ds(0, NPB)], gsem).wait()
            for g in range(g_rows):
                pltpu.make_async_copy(
                    emb_hbm.at[nbr_v.at[g]], nb_rows.at[g], gsem).wait()
                pltpu.make_async_copy(
                    p_hbm.at[nbr_v.at[g]], pnb_v.at[g], gsem).wait()

        def out_drain(blk, ds):
            (src_rows, nb_rows, qsrc_v, psrc_v, pnb_v, out_v, gsem,
             osem) = ds
            pltpu.make_async_copy(
                out_v, out_hbm.at[pl.ds(blk * NPB, NPB)], osem).wait()

        def compute(blk, ds, xs, first):
            nid_v, nbr_v, mask_v, isem = xs
            (src_rows, nb_rows, qsrc_v, psrc_v, pnb_v, out_v, gsem,
             osem) = ds
            nbase = blk * NPB
            qv = qsrc_v[...]
            pv = psrc_v[...]

            @pl.when(jnp.logical_not(first))
            def _():
                out_drain(blk, ds)

            def node_body(n, _):
                nfull = jnp.full((LANES,), n, jnp.int32)
                q_s = qv.at[nfull].get(mode="promise_in_bounds")
                p_s = pv.at[nfull].get(mode="promise_in_bounds")
                s_self = q_s + p_s
                s_self = jnp.where(s_self >= 0, s_self, 0.2 * s_self)

                svecs = []
                for gidx in range(m // LANES):
                    flat = n * m + gidx * LANES
                    grow = flat // 128
                    roff = pl.multiple_of(flat % 128, LANES)
                    p_nb = pnb_v[grow, pl.ds(roff, LANES)]
                    s = q_s + p_nb
                    s = jnp.where(s >= 0, s, 0.2 * s)
                    msk = mask_v[grow, pl.ds(roff, LANES)]
                    svecs.append(s + msk * (-1e9))

                smax = jnp.maximum(svecs[0], svecs[1])
                smax = jnp.maximum(smax, s_self)
                mval = allmax(smax)
                e0 = jnp.exp(svecs[0] - mval)
                e1 = jnp.exp(svecs[1] - mval)
                e_self = jnp.exp(s_self - mval)
                e_self_one = jnp.where(iota == 0, e_self, 0.0)
                denom = allsum(e0 + e1 + e_self_one)
                inv = 1.0 / denom
                ws = (e0 * inv, e1 * inv)
                w_self = e_self * inv

                acc = tuple(
                    w_self * src_rows[n, pl.ds(c * LANES, LANES)]
                    for c in range(hc))

                for gidx in range(m // LANES):
                    wg = ws[gidx]

                    def nb_body(j, acc, gidx=gidx, wg=wg):
                        w_j = wg.at[jnp.full((LANES,), j, jnp.int32)].get(
                            mode="promise_in_bounds")
                        flat = n * m + gidx * LANES + j
                        grow = flat // 128
                        roff = flat % 128
                        return tuple(
                            acc[c] + w_j * nb_rows[grow, roff,
                                                   pl.ds(c * LANES, LANES)]
                            for c in range(hc))

                    acc = lax.fori_loop(0, LANES, nb_body, acc, unroll=2)
                for c in range(hc):
                    out_v[n, pl.ds(c * LANES, LANES)] = acc[c]
                return 0

            lax.fori_loop(0, NPB, node_body, 0)
            pltpu.async_copy(out_v, out_hbm.at[pl.ds(nbase, NPB)], osem)

        for kk in range(4):
            idx_fire(wbase + kk, xsets[kk])
        idx_wait(xsets[0])
        g_fire(xsets[0], dsets[0])
        idx_wait(xsets[1])
        g_fire(xsets[1], dsets[1])

        def loop_body(i, _):
            b0 = wbase + 4 * i
            for k2 in range(4):
                ds = dsets[k2 % 2]
                xc = xsets[k2]
                xn = xsets[(k2 + 2) % 4]
                blk = b0 + k2
                g_wait(xc, ds)
                first = (i == 0) if k2 < 2 else (i < 0)
                compute(blk, ds, xc, first)
                idx_wait(xn)
                g_fire(xn, ds)
                idx_fire(blk + 4, xc)
            return 0

        lax.fori_loop(0, blocks_per_w // 4, loop_body, 0)

        g_wait(xsets[2], dsets[0])
        g_wait(xsets[3], dsets[1])
        idx_wait(xsets[2])
        idx_wait(xsets[3])
        out_drain(wbase + blocks_per_w - 2, dsets[0])
        out_drain(wbase + blocks_per_w - 1, dsets[1])

    return k(nid_flat, neigh_b, mask_b, emb, q1, p1)


def kernel(node_ids, neighs, mask, emb_table, a_w, a_b):
    b, l = node_ids.shape
    m = neighs.shape[-1]
    h = emb_table.shape[1]
    nt = b * l

    aw2 = a_w.reshape(2, h)
    bhalf = (a_b * 0.5).astype(jnp.float32)
    qp2 = _qp_tc(emb_table.astype(jnp.float32), aw2.astype(jnp.float32),
                 bhalf)
    q1 = qp2[0]
    p1 = qp2[1]

    nid_flat = node_ids.reshape(nt).astype(jnp.int32)
    gb = (NPB * m) // 128
    neigh_b = neighs.reshape(nt // NPB, gb, 128).astype(jnp.int32)
    mask_b = mask.reshape(nt // NPB, gb, 128).astype(jnp.float32)

    out = _gat_sc(nid_flat, neigh_b, mask_b, emb_table.astype(jnp.float32),
                  q1, p1, m)
    return out.reshape(b, l, h)


# speedup vs baseline: 1.5381x; 1.0246x over previous
import functools

import jax
import jax.numpy as jnp
from jax import lax
from jax.experimental import pallas as pl
from jax.experimental.pallas import tpu as pltpu
from jax.experimental.pallas import tpu_sc as plsc

LANES = 16
NPB = 8


def _qp_tc(emb, aw2, bhalf):
    rows, h = emb.shape
    blk = 12544
    grid = (rows + blk - 1) // blk

    def body(emb_ref, aw_ref, b_ref, out_ref):
        out_ref[...] = lax.dot_general(
            aw_ref[...], emb_ref[...],
            dimension_numbers=(((1,), (1,)), ((), ())),
            preferred_element_type=jnp.float32,
        ) + b_ref[0]

    return pl.pallas_call(
        body,
        grid=(grid,),
        in_specs=[
            pl.BlockSpec((blk, h), lambda i: (i, 0)),
            pl.BlockSpec((2, h), lambda i: (0, 0)),
            pl.BlockSpec(memory_space=pltpu.SMEM),
        ],
        out_specs=pl.BlockSpec((2, blk), lambda i: (0, i)),
        out_shape=jax.ShapeDtypeStruct((2, rows), jnp.float32),
    )(emb, aw2, bhalf)


def _gat_sc(cmb, emb, qpflat, p1, m):
    nblocks = cmb.shape[0]
    nt = nblocks * NPB
    h = emb.shape[1]
    v_rows = emb.shape[0]
    hc = h // LANES
    nbr_pb = NPB * m
    g_rows = nbr_pb // 128

    nc, ns = 2, 16
    nw = nc * ns
    blocks_per_w = nblocks // nw
    mesh = plsc.VectorSubcoreMesh(core_axis_name="c", subcore_axis_name="s",
                                  num_cores=nc, num_subcores=ns)

    pg = 2 * g_rows + 1

    idx_types = [
        pltpu.VMEM((pg, 128), jnp.int32),
        pltpu.SemaphoreType.DMA,
    ]
    data_types = [
        pltpu.VMEM((NPB, h), jnp.float32),
        pltpu.VMEM((nbr_pb, h), jnp.float32),
        pltpu.VMEM((2 * LANES,), jnp.float32),
        pltpu.VMEM((nbr_pb,), jnp.float32),
        pltpu.VMEM((NPB, h), jnp.float32),
        pltpu.SemaphoreType.DMA,
        pltpu.SemaphoreType.DMA,
    ]

    @functools.partial(
        pl.kernel,
        out_type=jax.ShapeDtypeStruct((nt, h), jnp.float32),
        mesh=mesh,
        scratch_types=[[idx_types] * 4, [data_types] * 2],
    )
    def k(cmb_hbm, emb_hbm, qpflat_hbm, p_hbm, out_hbm, xsets, dsets):
        wid = lax.axis_index("s") * nc + lax.axis_index("c")
        iota = lax.iota(jnp.int32, LANES)
        wbase = wid * blocks_per_w

        def _shuf(x, sh):
            return x.at[iota ^ sh].get(mode="promise_in_bounds")

        def allmax(x):
            for sh in (8, 4, 2, 1):
                x = jnp.maximum(x, _shuf(x, sh))
            return x

        def allsum(x):
            for sh in (8, 4, 2, 1):
                x = x + _shuf(x, sh)
            return x

        def idx_fire(blk, xs):
            cmb_v, isem = xs
            blk = jnp.minimum(blk, nblocks - 1)
            pltpu.async_copy(cmb_hbm.at[blk], cmb_v, isem)

        def idx_wait(xs):
            cmb_v, isem = xs
            pltpu.make_async_copy(cmb_hbm.at[0], cmb_v, isem).wait()

        def g_fire(xs, ds):
            cmb_v, isem = xs
            src_rows, nb_rows, qp16_v, pnb_v, out_v, gsem, osem = ds
            nidv = cmb_v[g_rows, pl.ds(0, LANES)]
            comb = jnp.where(iota < NPB, nidv, _shuf(nidv, 8) + v_rows)
            pltpu.async_copy(qpflat_hbm.at[comb], qp16_v.at[pl.ds(0, LANES)], gsem)
            pltpu.async_copy(emb_hbm.at[cmb_v.at[g_rows, pl.ds(0, NPB)]],
                             src_rows, gsem)
            for g in range(g_rows):
                pltpu.async_copy(emb_hbm.at[cmb_v.at[g]],
                                 nb_rows.at[pl.ds(g * 128, 128)], gsem)
                pltpu.async_copy(p_hbm.at[cmb_v.at[g]],
                                 pnb_v.at[pl.ds(g * 128, 128)], gsem)

        def g_wait(xs, ds):
            cmb_v, isem = xs
            src_rows, nb_rows, qp16_v, pnb_v, out_v, gsem, osem = ds
            pltpu.make_async_copy(
                qpflat_hbm.at[iota], qp16_v.at[pl.ds(0, LANES)], gsem).wait()
            pltpu.make_async_copy(
                emb_hbm.at[cmb_v.at[g_rows, pl.ds(0, NPB)]], src_rows,
                gsem).wait()
            for g in range(g_rows):
                pltpu.make_async_copy(
                    emb_hbm.at[cmb_v.at[g]],
                    nb_rows.at[pl.ds(g * 128, 128)], gsem).wait()
                pltpu.make_async_copy(
                    p_hbm.at[cmb_v.at[g]],
                    pnb_v.at[pl.ds(g * 128, 128)], gsem).wait()

        def out_drain(blk, ds):
            src_rows, nb_rows, qp16_v, pnb_v, out_v, gsem, osem = ds
            pltpu.make_async_copy(
                out_v, out_hbm.at[pl.ds(blk * NPB, NPB)], osem).wait()

        def compute(blk, ds, xs, first):
            cmb_v, isem = xs
            src_rows, nb_rows, qp16_v, pnb_v, out_v, gsem, osem = ds
            nbase = blk * NPB
            qp16 = qp16_v[pl.ds(0, LANES)]

            @pl.when(jnp.logical_not(first))
            def _():
                out_drain(blk, ds)

            def node_body(n, _):
                nfull = jnp.full((LANES,), n, jnp.int32)
                q_s = qp16.at[nfull].get(mode="promise_in_bounds")
                p_s = qp16.at[nfull + NPB].get(mode="promise_in_bounds")
                s_self = q_s + p_s
                s_self = jnp.where(s_self >= 0, s_self, 0.2 * s_self)

                svecs = []
                for gidx in range(m // LANES):
                    flat = n * m + gidx * LANES
                    roff = pl.multiple_of(flat, LANES)
                    p_nb = pnb_v[pl.ds(roff, LANES)]
                    s = q_s + p_nb
                    s = jnp.where(s >= 0, s, 0.2 * s)
                    mrow = (g_rows + 1) + flat // 128
                    moff = pl.multiple_of(flat % 128, LANES)
                    msk = cmb_v[mrow, pl.ds(moff, LANES)]
                    svecs.append(jnp.where(msk != 0, s - 1e9, s))

                smax = jnp.maximum(svecs[0], svecs[1])
                smax = jnp.maximum(smax, s_self)
                mval = allmax(smax)
                e0 = jnp.exp(svecs[0] - mval)
                e1 = jnp.exp(svecs[1] - mval)
                e_self = jnp.exp(s_self - mval)
                e_self_one = jnp.where(iota == 0, e_self, 0.0)
                denom = allsum(e0 + e1 + e_self_one)
                inv = 1.0 / denom
                ws = (e0 * inv, e1 * inv)
                w_self = e_self * inv

                acc = tuple(
                    w_self * src_rows[n, pl.ds(c * LANES, LANES)]
                    for c in range(hc))

                for gidx in range(m // LANES):
                    wg = ws[gidx]

                    def nb_body(j, acc, gidx=gidx, wg=wg):
                        w_j = wg.at[jnp.full((LANES,), j, jnp.int32)].get(
                            mode="promise_in_bounds")
                        flat = n * m + gidx * LANES + j
                        return tuple(
                            acc[c] + w_j * nb_rows[flat,
                                                   pl.ds(c * LANES, LANES)]
                            for c in range(hc))

                    acc = lax.fori_loop(0, LANES, nb_body, acc, unroll=2)
                for c in range(hc):
                    out_v[n, pl.ds(c * LANES, LANES)] = acc[c]
                return 0

            lax.fori_loop(0, NPB, node_body, 0)
            pltpu.async_copy(out_v, out_hbm.at[pl.ds(nbase, NPB)], osem)

        for kk in range(4):
            idx_fire(wbase + kk, xsets[kk])
        idx_wait(xsets[0])
        g_fire(xsets[0], dsets[0])
        idx_wait(xsets[1])
        g_fire(xsets[1], dsets[1])

        def loop_body(i, _):
            b0 = wbase + 4 * i
            for k2 in range(4):
                ds = dsets[k2 % 2]
                xc = xsets[k2]
                xn = xsets[(k2 + 2) % 4]
                blk = b0 + k2
                g_wait(xc, ds)
                first = (i == 0) if k2 < 2 else (i < 0)
                compute(blk, ds, xc, first)
                idx_wait(xn)
                g_fire(xn, ds)
                idx_fire(blk + 4, xc)
            return 0

        lax.fori_loop(0, blocks_per_w // 4, loop_body, 0)

        g_wait(xsets[2], dsets[0])
        g_wait(xsets[3], dsets[1])
        idx_wait(xsets[2])
        idx_wait(xsets[3])
        out_drain(wbase + blocks_per_w - 2, dsets[0])
        out_drain(wbase + blocks_per_w - 1, dsets[1])

    return k(cmb, emb, qpflat, p1)


def kernel(node_ids, neighs, mask, emb_table, a_w, a_b):
    b, l = node_ids.shape
    m = neighs.shape[-1]
    h = emb_table.shape[1]
    nt = b * l
    nblocks = nt // NPB
    g_rows = (NPB * m) // 128

    aw2 = a_w.reshape(2, h)
    bhalf = (a_b * 0.5).astype(jnp.float32)
    qp2 = _qp_tc(emb_table.astype(jnp.float32), aw2.astype(jnp.float32),
                 bhalf)
    qpflat = qp2.reshape(2 * qp2.shape[1])
    p1 = qp2[1]

    nbr_rows = neighs.reshape(nblocks, g_rows, 128).astype(jnp.int32)
    nid_row = jnp.pad(node_ids.reshape(nblocks, NPB).astype(jnp.int32),
                      ((0, 0), (0, 128 - NPB)))[:, None, :]
    mask_rows = mask.reshape(nblocks, g_rows, 128).astype(jnp.int32)
    cmb = jnp.concatenate([nbr_rows, nid_row, mask_rows], axis=1)

    out = _gat_sc(cmb, emb_table.astype(jnp.float32), qpflat, p1, m)
    return out.reshape(b, l, h)


# speedup vs baseline: 1.5647x; 1.0173x over previous
import functools

import jax
import jax.numpy as jnp
from jax import lax
from jax.experimental import pallas as pl
from jax.experimental.pallas import tpu as pltpu
from jax.experimental.pallas import tpu_sc as plsc

LANES = 16
NPB = 8


def _qp_tc(emb, aw2, bhalf):
    rows, h = emb.shape
    blk = 12544
    grid = (rows + blk - 1) // blk

    def body(emb_ref, aw_ref, b_ref, out_ref):
        out_ref[...] = lax.dot_general(
            aw_ref[...], emb_ref[...],
            dimension_numbers=(((1,), (1,)), ((), ())),
            preferred_element_type=jnp.float32,
        ) + b_ref[0]

    return pl.pallas_call(
        body,
        grid=(grid,),
        in_specs=[
            pl.BlockSpec((blk, h), lambda i: (i, 0)),
            pl.BlockSpec((2, h), lambda i: (0, 0)),
            pl.BlockSpec(memory_space=pltpu.SMEM),
        ],
        out_specs=pl.BlockSpec((2, blk), lambda i: (0, i)),
        out_shape=jax.ShapeDtypeStruct((2, rows), jnp.float32),
    )(emb, aw2, bhalf)


def _gat_sc(nid_flat, neigh_b, mask_b, emb, q1, p1, m):
    nt = nid_flat.shape[0]
    h = emb.shape[1]
    hc = h // LANES
    nb_rows_per_block = NPB * m
    g_rows = nb_rows_per_block // 128

    nc, ns = 2, 16
    nw = nc * ns
    nblocks = nt // NPB
    blocks_per_w = nblocks // nw
    mesh = plsc.VectorSubcoreMesh(core_axis_name="c", subcore_axis_name="s",
                                  num_cores=nc, num_subcores=ns)

    idx_types = [
        pltpu.VMEM((NPB,), jnp.int32),
        pltpu.VMEM((g_rows, 128), jnp.int32),
        pltpu.VMEM((g_rows, 128), jnp.float32),
        pltpu.SemaphoreType.DMA,
    ]
    data_types = [
        pltpu.VMEM((NPB, h), jnp.float32),
        pltpu.VMEM((g_rows, 128, h), jnp.float32),
        pltpu.VMEM((LANES,), jnp.float32),
        pltpu.VMEM((LANES,), jnp.float32),
        pltpu.VMEM((g_rows, 128), jnp.float32),
        pltpu.VMEM((NPB, h), jnp.float32),
        pltpu.SemaphoreType.DMA,
        pltpu.SemaphoreType.DMA,
    ]

    @functools.partial(
        pl.kernel,
        out_type=jax.ShapeDtypeStruct((nt, h), jnp.float32),
        mesh=mesh,
        scratch_types=[[idx_types] * 4, [data_types] * 2],
    )
    def k(nid_hbm, neigh_hbm, mask_hbm, emb_hbm, q_hbm, p_hbm, out_hbm,
          xsets, dsets):
        wid = lax.axis_index("s") * nc + lax.axis_index("c")
        iota = lax.iota(jnp.int32, LANES)
        wbase = wid * blocks_per_w

        def _shuf(x, sh):
            return x.at[iota ^ sh].get(mode="promise_in_bounds")

        def allmax(x):
            for sh in (8, 4, 2, 1):
                x = jnp.maximum(x, _shuf(x, sh))
            return x

        def allsum(x):
            for sh in (8, 4, 2, 1):
                x = x + _shuf(x, sh)
            return x

        def idx_fire(blk, xs):
            nid_v, nbr_v, mask_v, isem = xs
            blk = jnp.minimum(blk, nblocks - 1)
            nbase = blk * NPB
            pltpu.async_copy(nid_hbm.at[pl.ds(nbase, NPB)], nid_v, isem)
            pltpu.async_copy(neigh_hbm.at[blk], nbr_v, isem)
            pltpu.async_copy(mask_hbm.at[blk], mask_v, isem)

        def idx_wait(xs):
            nid_v, nbr_v, mask_v, isem = xs
            pltpu.make_async_copy(nid_hbm.at[pl.ds(0, NPB)], nid_v, isem).wait()
            pltpu.make_async_copy(neigh_hbm.at[0], nbr_v, isem).wait()
            pltpu.make_async_copy(mask_hbm.at[0], mask_v, isem).wait()

        def g_fire(xs, ds):
            nid_v, nbr_v, mask_v, isem = xs
            (src_rows, nb_rows, qsrc_v, psrc_v, pnb_v, out_v, gsem,
             osem) = ds
            pltpu.async_copy(emb_hbm.at[nid_v], src_rows, gsem)
            pltpu.async_copy(q_hbm.at[nid_v], qsrc_v.at[pl.ds(0, NPB)], gsem)
            pltpu.async_copy(p_hbm.at[nid_v], psrc_v.at[pl.ds(0, NPB)], gsem)
            for g in range(g_rows):
                pltpu.async_copy(emb_hbm.at[nbr_v.at[g]], nb_rows.at[g], gsem)
                pltpu.async_copy(p_hbm.at[nbr_v.at[g]], pnb_v.at[g], gsem)

        def g_wait(xs, ds):
            nid_v, nbr_v, mask_v, isem = xs
            (src_rows, nb_rows, qsrc_v, psrc_v, pnb_v, out_v, gsem,
             osem) = ds
            pltpu.make_async_copy(emb_hbm.at[nid_v], src_rows, gsem).wait()
            pltpu.make_async_copy(
                q_hbm.at[nid_v], qsrc_v.at[pl.ds(0, NPB)], gsem).wait()
            pltpu.make_async_copy(
                p_hbm.at[nid_v], psrc_v.at[pl.ds(0, NPB)], gsem).wait()
            for g in range(g_rows):
                pltpu.make_async_copy(
                    emb_hbm.at[nbr_v.at[g]], nb_rows.at[g], gsem).wait()
                pltpu.make_async_copy(
                    p_hbm.at[nbr_v.at[g]], pnb_v.at[g], gsem).wait()

        def out_drain(blk, ds):
            (src_rows, nb_rows, qsrc_v, psrc_v, pnb_v, out_v, gsem,
             osem) = ds
            pltpu.make_async_copy(
                out_v, out_hbm.at[pl.ds(blk * NPB, NPB)], osem).wait()

        def compute(blk, ds, xs, first):
            nid_v, nbr_v, mask_v, isem = xs
            (src_rows, nb_rows, qsrc_v, psrc_v, pnb_v, out_v, gsem,
             osem) = ds
            nbase = blk * NPB
            qv = qsrc_v[...]
            pv = psrc_v[...]

            @pl.when(jnp.logical_not(first))
            def _():
                out_drain(blk, ds)

            def node_body(n, _):
                nfull = jnp.full((LANES,), n, jnp.int32)
                q_s = qv.at[nfull].get(mode="promise_in_bounds")
                p_s = pv.at[nfull].get(mode="promise_in_bounds")
                s_self = q_s + p_s
                s_self = jnp.where(s_self >= 0, s_self, 0.2 * s_self)

                svecs = []
                for gidx in range(m // LANES):
                    flat = n * m + gidx * LANES
                    grow = flat // 128
                    roff = pl.multiple_of(flat % 128, LANES)
                    p_nb = pnb_v[grow, pl.ds(roff, LANES)]
                    s = q_s + p_nb
                    s = jnp.where(s >= 0, s, 0.2 * s)
                    msk = mask_v[grow, pl.ds(roff, LANES)]
                    svecs.append(s + msk * (-1e9))

                smax = jnp.maximum(svecs[0], svecs[1])
                smax = jnp.maximum(smax, s_self)
                mval = allmax(smax)
                e0 = jnp.exp(svecs[0] - mval)
                e1 = jnp.exp(svecs[1] - mval)
                e_self = jnp.exp(s_self - mval)
                e_self_one = jnp.where(iota == 0, e_self, 0.0)
                denom = allsum(e0 + e1 + e_self_one)
                inv = 1.0 / denom
                ws = (e0 * inv, e1 * inv)
                w_self = e_self * inv

                acc = tuple(
                    w_self * src_rows[n, pl.ds(c * LANES, LANES)]
                    for c in range(hc))

                for gidx in range(m // LANES):
                    wg = ws[gidx]

                    def nb_body(j, acc, gidx=gidx, wg=wg):
                        w_j = wg.at[jnp.full((LANES,), j, jnp.int32)].get(
                            mode="promise_in_bounds")
                        flat = n * m + gidx * LANES + j
                        grow = flat // 128
                        roff = flat % 128
                        return tuple(
                            acc[c] + w_j * nb_rows[grow, roff,
                                                   pl.ds(c * LANES, LANES)]
                            for c in range(hc))

                    acc = lax.fori_loop(0, LANES, nb_body, acc, unroll=4)
                for c in range(hc):
                    out_v[n, pl.ds(c * LANES, LANES)] = acc[c]
                return 0

            lax.fori_loop(0, NPB, node_body, 0)
            pltpu.async_copy(out_v, out_hbm.at[pl.ds(nbase, NPB)], osem)

        for kk in range(4):
            idx_fire(wbase + kk, xsets[kk])
        idx_wait(xsets[0])
        g_fire(xsets[0], dsets[0])
        idx_wait(xsets[1])
        g_fire(xsets[1], dsets[1])

        def loop_body(i, _):
            b0 = wbase + 4 * i
            for k2 in range(4):
                ds = dsets[k2 % 2]
                xc = xsets[k2]
                xn = xsets[(k2 + 2) % 4]
                blk = b0 + k2
                g_wait(xc, ds)
                first = (i == 0) if k2 < 2 else (i < 0)
                compute(blk, ds, xc, first)
                idx_wait(xn)
                g_fire(xn, ds)
                idx_fire(blk + 4, xc)
            return 0

        lax.fori_loop(0, blocks_per_w // 4, loop_body, 0)

        g_wait(xsets[2], dsets[0])
        g_wait(xsets[3], dsets[1])
        idx_wait(xsets[2])
        idx_wait(xsets[3])
        out_drain(wbase + blocks_per_w - 2, dsets[0])
        out_drain(wbase + blocks_per_w - 1, dsets[1])

    return k(nid_flat, neigh_b, mask_b, emb, q1, p1)


def kernel(node_ids, neighs, mask, emb_table, a_w, a_b):
    b, l = node_ids.shape
    m = neighs.shape[-1]
    h = emb_table.shape[1]
    nt = b * l

    aw2 = a_w.reshape(2, h)
    bhalf = (a_b * 0.5).astype(jnp.float32)
    qp2 = _qp_tc(emb_table.astype(jnp.float32), aw2.astype(jnp.float32),
                 bhalf)
    q1 = qp2[0]
    p1 = qp2[1]

    nid_flat = node_ids.reshape(nt).astype(jnp.int32)
    gb = (NPB * m) // 128
    neigh_b = neighs.reshape(nt // NPB, gb, 128).astype(jnp.int32)
    mask_b = mask.reshape(nt // NPB, gb, 128).astype(jnp.float32)

    out = _gat_sc(nid_flat, neigh_b, mask_b, emb_table.astype(jnp.float32),
                  q1, p1, m)
    return out.reshape(b, l, h)
